# Initial kernel scaffold; baseline (speedup 1.0000x reference)
import jax
import jax.numpy as jnp
from jax.experimental import pallas as pl


def kernel(features, edge_index, W_self0, W_neigh0, b0, W_self1, W_neigh1, b1):
    raise NotImplementedError("write your pallas kernel here")


# speedup vs baseline: 3.9006x; 3.9006x over previous
import jax
import jax.numpy as jnp
from jax import lax
from jax.experimental import pallas as pl
from jax.experimental.pallas import tpu as pltpu
from jax.experimental.pallas import tpu_sc as plsc

N = 10000
D = 128
H = 128
C_OUT = 47
E = 320000
W2 = 64

NC = 2
NS = 16
NW = NC * NS
G = 128
RPW = 80
E_PAD = NW * RPW * G
N_PAD = 10240
RT = N_PAD // NS
MCH = 8
NM = RPW // MCH


def _fill(ref, nrows, ncols, value):
    v = jnp.full((16,), value, dtype=jnp.float32)
    nchunk = ncols // 16

    def body(i, carry):
        ref[i // nchunk, pl.ds((i % nchunk) * 16, 16)] = v
        return carry

    lax.fori_loop(0, nrows * nchunk, body, 0)


def _sc_agg(table, src2, dst2, with_deg):
    width = table.shape[1]

    def body(*refs):
        if with_deg:
            (src2_h, dst2_h, table_h, acc_o, deg_o,
             src_v, dst_v, rows_v, acc_sh, sem, ones_v, deg_sh) = refs
        else:
            (src2_h, dst2_h, table_h, acc_o,
             src_v, dst_v, rows_v, acc_sh, sem) = refs
        cid = lax.axis_index("c")
        sid = lax.axis_index("s")
        wid = cid * NS + sid

        _fill(rows_v, G, width, 0.0)
        for k in range(RT // G):
            pltpu.sync_copy(rows_v, acc_sh.at[pl.ds(sid * RT + k * G, G)])
        if with_deg:
            _fill(ones_v, G, 16, 0.0)
            for k in range(RT // G):
                pltpu.sync_copy(ones_v, deg_sh.at[pl.ds(sid * RT + k * G, G)])
            _fill(ones_v, G, 16, 1.0)

        plsc.subcore_barrier()

        def macro(m, carry):
            off = pl.multiple_of(wid * RPW + m * MCH, 8)
            pltpu.sync_copy(src2_h.at[pl.ds(off, MCH)], src_v)
            pltpu.sync_copy(dst2_h.at[pl.ds(off, MCH)], dst_v)

            def group(j, c2):
                pltpu.async_copy(table_h.at[src_v.at[j]], rows_v, sem).wait()
                pltpu.sync_copy(rows_v, acc_sh.at[dst_v.at[j]], add=True)
                if with_deg:
                    pltpu.sync_copy(ones_v, deg_sh.at[dst_v.at[j]], add=True)
                return c2

            lax.fori_loop(0, MCH, group, 0)
            return carry

        lax.fori_loop(0, NM, macro, 0)

        plsc.subcore_barrier()

        pltpu.sync_copy(acc_sh.at[pl.ds(sid * RT, RT)],
                        acc_o.at[cid, pl.ds(sid * RT, RT)])
        if with_deg:
            pltpu.sync_copy(deg_sh.at[pl.ds(sid * RT, RT)],
                            deg_o.at[cid, pl.ds(sid * RT, RT)])

    out_type = [jax.ShapeDtypeStruct((NC, N_PAD, width), jnp.float32)]
    scratch = [pltpu.VMEM((MCH, G), jnp.int32),
               pltpu.VMEM((MCH, G), jnp.int32),
               pltpu.VMEM((G, width), jnp.float32),
               pltpu.VMEM_SHARED((N_PAD, width), jnp.float32),
               pltpu.SemaphoreType.DMA]
    if with_deg:
        out_type.append(jax.ShapeDtypeStruct((NC, N_PAD, 16), jnp.float32))
        scratch += [pltpu.VMEM((G, 16), jnp.float32),
                    pltpu.VMEM_SHARED((N_PAD, 16), jnp.float32)]

    fn = pl.kernel(
        body,
        out_type=out_type,
        mesh=plsc.VectorSubcoreMesh(core_axis_name="c", subcore_axis_name="s"),
        scratch_types=scratch,
        compiler_params=pltpu.CompilerParams(use_tc_tiling_on_sc=False),
    )
    return fn(src2, dst2, table)


R = 400
NB = N // R


def _tc_layer0(x, wn, ws, b):
    def body(x_r, wn_r, ws_r, b_r, xw_o, self_o):
        xv = x_r[...]
        xw_o[...] = jnp.dot(xv, wn_r[...], preferred_element_type=jnp.float32)
        self_o[...] = jnp.dot(xv, ws_r[...],
                              preferred_element_type=jnp.float32) + b_r[...]

    return pl.pallas_call(
        body,
        grid=(NB,),
        in_specs=[pl.BlockSpec((R, D), lambda i: (i, 0)),
                  pl.BlockSpec((D, H), lambda i: (0, 0)),
                  pl.BlockSpec((D, H), lambda i: (0, 0)),
                  pl.BlockSpec((1, H), lambda i: (0, 0))],
        out_specs=[pl.BlockSpec((R, H), lambda i: (i, 0)),
                   pl.BlockSpec((R, H), lambda i: (i, 0))],
        out_shape=[jax.ShapeDtypeStruct((N, H), jnp.float32),
                   jax.ShapeDtypeStruct((N, H), jnp.float32)],
    )(x, wn, ws, b.reshape(1, H))


def _tc_mid(self0, acc, deg, wn1p, ws1p, b1p):
    def body(s_r, a_r, d_r, wn_r, ws_r, b_r, hw_o, hs_o):
        degv = d_r[0, :, 0:1] + d_r[1, :, 0:1]
        accv = a_r[0] + a_r[1]
        h = s_r[...] + accv / jnp.maximum(degv, 1.0)
        h = jnp.maximum(h, 0.0)
        nrm = jnp.sqrt(jnp.sum(h * h, axis=1, keepdims=True))
        h = h / jnp.maximum(nrm, 1e-12)
        hw_o[...] = jnp.dot(h, wn_r[...], preferred_element_type=jnp.float32)
        hs_o[...] = jnp.dot(h, ws_r[...],
                            preferred_element_type=jnp.float32) + b_r[...]

    return pl.pallas_call(
        body,
        grid=(NB,),
        in_specs=[pl.BlockSpec((R, H), lambda i: (i, 0)),
                  pl.BlockSpec((NC, R, H), lambda i: (0, i, 0)),
                  pl.BlockSpec((NC, R, 16), lambda i: (0, i, 0)),
                  pl.BlockSpec((H, W2), lambda i: (0, 0)),
                  pl.BlockSpec((H, W2), lambda i: (0, 0)),
                  pl.BlockSpec((1, W2), lambda i: (0, 0))],
        out_specs=[pl.BlockSpec((R, W2), lambda i: (i, 0)),
                   pl.BlockSpec((R, W2), lambda i: (i, 0))],
        out_shape=[jax.ShapeDtypeStruct((N, W2), jnp.float32),
                   jax.ShapeDtypeStruct((N, W2), jnp.float32)],
    )(self0, acc, deg, wn1p, ws1p, b1p)


def _tc_final(hs1, acc, deg):
    def body(s_r, a_r, d_r, o_r):
        degv = d_r[0, :, 0:1] + d_r[1, :, 0:1]
        o_r[...] = s_r[...] + (a_r[0] + a_r[1]) / jnp.maximum(degv, 1.0)

    return pl.pallas_call(
        body,
        grid=(NB,),
        in_specs=[pl.BlockSpec((R, W2), lambda i: (i, 0)),
                  pl.BlockSpec((NC, R, W2), lambda i: (0, i, 0)),
                  pl.BlockSpec((NC, R, 16), lambda i: (0, i, 0))],
        out_specs=pl.BlockSpec((R, W2), lambda i: (i, 0)),
        out_shape=jax.ShapeDtypeStruct((N, W2), jnp.float32),
    )(hs1, acc, deg)


def kernel(features, edge_index, W_self0, W_neigh0, b0, W_self1, W_neigh1, b1):
    pad = E_PAD - E
    pad_src = jnp.zeros((pad,), jnp.int32)
    pad_dst = N + jnp.arange(pad, dtype=jnp.int32) % (N_PAD - N)
    src2 = jnp.concatenate([edge_index[0], pad_src]).reshape(E_PAD // G, G)
    dst2 = jnp.concatenate([edge_index[1], pad_dst]).reshape(E_PAD // G, G)
    wn1p = jnp.zeros((H, W2), jnp.float32).at[:, :C_OUT].set(W_neigh1)
    ws1p = jnp.zeros((H, W2), jnp.float32).at[:, :C_OUT].set(W_self1)
    b1p = jnp.zeros((1, W2), jnp.float32).at[0, :C_OUT].set(b1)

    xw0, self0 = _tc_layer0(features, W_neigh0, W_self0, b0)
    acc0, deg = _sc_agg(xw0, src2, dst2, with_deg=True)
    hw1, hs1 = _tc_mid(self0, acc0, deg, wn1p, ws1p, b1p)
    (acc1,) = _sc_agg(hw1, src2, dst2, with_deg=False)
    out64 = _tc_final(hs1, acc1, deg)
    return out64[:, :C_OUT]


# speedup vs baseline: 4.4166x; 1.1323x over previous
import jax
import jax.numpy as jnp
from jax import lax
from jax.experimental import pallas as pl
from jax.experimental.pallas import tpu as pltpu
from jax.experimental.pallas import tpu_sc as plsc

N = 10000
D = 128
H = 128
C_OUT = 47
E = 320000
W2 = 64

NC = 2
NS = 16
NW = NC * NS
G = 128
RPW = 80
E_PAD = NW * RPW * G
N_PAD = 10240
RT = N_PAD // NS
MCH = 8
NM = RPW // MCH


def _fill(ref, nrows, ncols, value):
    v = jnp.full((16,), value, dtype=jnp.float32)
    nchunk = ncols // 16

    def body(i, carry):
        ref[i // nchunk, pl.ds((i % nchunk) * 16, 16)] = v
        return carry

    lax.fori_loop(0, nrows * nchunk, body, 0)


def _sc_agg(table, src2, dst2, with_deg):
    width = table.shape[1]

    def body(*refs):
        if with_deg:
            (src2_h, dst2_h, table_h, acc_o, deg_o,
             src_v, dst_v, rows_v, acc_sh, sem0, sem1, ones_v, deg_sh) = refs
        else:
            (src2_h, dst2_h, table_h, acc_o,
             src_v, dst_v, rows_v, acc_sh, sem0, sem1) = refs
        cid = lax.axis_index("c")
        sid = lax.axis_index("s")
        wid = cid * NS + sid

        _fill(rows_v.at[0], G, width, 0.0)
        for k in range(RT // G):
            pltpu.sync_copy(rows_v.at[0], acc_sh.at[pl.ds(sid * RT + k * G, G)])
        if with_deg:
            _fill(ones_v, G, 16, 0.0)
            for k in range(RT // G):
                pltpu.sync_copy(ones_v, deg_sh.at[pl.ds(sid * RT + k * G, G)])
            _fill(ones_v, G, 16, 1.0)

        plsc.subcore_barrier()

        def macro(m, carry):
            off = pl.multiple_of(wid * RPW + m * MCH, 8)
            pltpu.sync_copy(src2_h.at[pl.ds(off, MCH)], src_v)
            pltpu.sync_copy(dst2_h.at[pl.ds(off, MCH)], dst_v)

            pltpu.async_copy(table_h.at[src_v.at[0]], rows_v.at[0], sem0)
            for j in range(MCH):
                p = j % 2
                psem = sem0 if p == 0 else sem1
                if j + 1 < MCH:
                    nsem = sem1 if p == 0 else sem0
                    pltpu.async_copy(table_h.at[src_v.at[j + 1]],
                                     rows_v.at[1 - p], nsem)
                pltpu.make_async_copy(table_h.at[src_v.at[j]],
                                      rows_v.at[p], psem).wait()
                pltpu.sync_copy(rows_v.at[p], acc_sh.at[dst_v.at[j]], add=True)
                if with_deg:
                    pltpu.sync_copy(ones_v, deg_sh.at[dst_v.at[j]], add=True)
            return carry

        lax.fori_loop(0, NM, macro, 0)

        plsc.subcore_barrier()

        pltpu.sync_copy(acc_sh.at[pl.ds(sid * RT, RT)],
                        acc_o.at[cid, pl.ds(sid * RT, RT)])
        if with_deg:
            pltpu.sync_copy(deg_sh.at[pl.ds(sid * RT, RT)],
                            deg_o.at[cid, pl.ds(sid * RT, RT)])

    out_type = [jax.ShapeDtypeStruct((NC, N_PAD, width), jnp.float32)]
    scratch = [pltpu.VMEM((MCH, G), jnp.int32),
               pltpu.VMEM((MCH, G), jnp.int32),
               pltpu.VMEM((2, G, width), jnp.float32),
               pltpu.VMEM_SHARED((N_PAD, width), jnp.float32),
               pltpu.SemaphoreType.DMA,
               pltpu.SemaphoreType.DMA]
    if with_deg:
        out_type.append(jax.ShapeDtypeStruct((NC, N_PAD, 16), jnp.float32))
        scratch += [pltpu.VMEM((G, 16), jnp.float32),
                    pltpu.VMEM_SHARED((N_PAD, 16), jnp.float32)]

    fn = pl.kernel(
        body,
        out_type=out_type,
        mesh=plsc.VectorSubcoreMesh(core_axis_name="c", subcore_axis_name="s"),
        scratch_types=scratch,
        compiler_params=pltpu.CompilerParams(use_tc_tiling_on_sc=False),
    )
    return fn(src2, dst2, table)


R = 400
NB = N // R


def _tc_layer0(x, wn, ws, b):
    def body(x_r, wn_r, ws_r, b_r, xw_o, self_o):
        xv = x_r[...]
        xw_o[...] = jnp.dot(xv, wn_r[...], preferred_element_type=jnp.float32)
        self_o[...] = jnp.dot(xv, ws_r[...],
                              preferred_element_type=jnp.float32) + b_r[...]

    return pl.pallas_call(
        body,
        grid=(NB,),
        in_specs=[pl.BlockSpec((R, D), lambda i: (i, 0)),
                  pl.BlockSpec((D, H), lambda i: (0, 0)),
                  pl.BlockSpec((D, H), lambda i: (0, 0)),
                  pl.BlockSpec((1, H), lambda i: (0, 0))],
        out_specs=[pl.BlockSpec((R, H), lambda i: (i, 0)),
                   pl.BlockSpec((R, H), lambda i: (i, 0))],
        out_shape=[jax.ShapeDtypeStruct((N, H), jnp.float32),
                   jax.ShapeDtypeStruct((N, H), jnp.float32)],
    )(x, wn, ws, b.reshape(1, H))


def _tc_mid(self0, acc, deg, wn1p, ws1p, b1p):
    def body(s_r, a_r, d_r, wn_r, ws_r, b_r, hw_o, hs_o):
        degv = d_r[0, :, 0:1] + d_r[1, :, 0:1]
        accv = a_r[0] + a_r[1]
        h = s_r[...] + accv / jnp.maximum(degv, 1.0)
        h = jnp.maximum(h, 0.0)
        nrm = jnp.sqrt(jnp.sum(h * h, axis=1, keepdims=True))
        h = h / jnp.maximum(nrm, 1e-12)
        hw_o[...] = jnp.dot(h, wn_r[...], preferred_element_type=jnp.float32)
        hs_o[...] = jnp.dot(h, ws_r[...],
                            preferred_element_type=jnp.float32) + b_r[...]

    return pl.pallas_call(
        body,
        grid=(NB,),
        in_specs=[pl.BlockSpec((R, H), lambda i: (i, 0)),
                  pl.BlockSpec((NC, R, H), lambda i: (0, i, 0)),
                  pl.BlockSpec((NC, R, 16), lambda i: (0, i, 0)),
                  pl.BlockSpec((H, W2), lambda i: (0, 0)),
                  pl.BlockSpec((H, W2), lambda i: (0, 0)),
                  pl.BlockSpec((1, W2), lambda i: (0, 0))],
        out_specs=[pl.BlockSpec((R, W2), lambda i: (i, 0)),
                   pl.BlockSpec((R, W2), lambda i: (i, 0))],
        out_shape=[jax.ShapeDtypeStruct((N, W2), jnp.float32),
                   jax.ShapeDtypeStruct((N, W2), jnp.float32)],
    )(self0, acc, deg, wn1p, ws1p, b1p)


def _tc_final(hs1, acc, deg):
    def body(s_r, a_r, d_r, o_r):
        degv = d_r[0, :, 0:1] + d_r[1, :, 0:1]
        o_r[...] = s_r[...] + (a_r[0] + a_r[1]) / jnp.maximum(degv, 1.0)

    return pl.pallas_call(
        body,
        grid=(NB,),
        in_specs=[pl.BlockSpec((R, W2), lambda i: (i, 0)),
                  pl.BlockSpec((NC, R, W2), lambda i: (0, i, 0)),
                  pl.BlockSpec((NC, R, 16), lambda i: (0, i, 0))],
        out_specs=pl.BlockSpec((R, W2), lambda i: (i, 0)),
        out_shape=jax.ShapeDtypeStruct((N, W2), jnp.float32),
    )(hs1, acc, deg)


def kernel(features, edge_index, W_self0, W_neigh0, b0, W_self1, W_neigh1, b1):
    pad = E_PAD - E
    pad_src = jnp.zeros((pad,), jnp.int32)
    pad_dst = N + jnp.arange(pad, dtype=jnp.int32) % (N_PAD - N)
    src2 = jnp.concatenate([edge_index[0], pad_src]).reshape(E_PAD // G, G)
    dst2 = jnp.concatenate([edge_index[1], pad_dst]).reshape(E_PAD // G, G)
    wn1p = jnp.zeros((H, W2), jnp.float32).at[:, :C_OUT].set(W_neigh1)
    ws1p = jnp.zeros((H, W2), jnp.float32).at[:, :C_OUT].set(W_self1)
    b1p = jnp.zeros((1, W2), jnp.float32).at[0, :C_OUT].set(b1)

    xw0, self0 = _tc_layer0(features, W_neigh0, W_self0, b0)
    acc0, deg = _sc_agg(xw0, src2, dst2, with_deg=True)
    hw1, hs1 = _tc_mid(self0, acc0, deg, wn1p, ws1p, b1p)
    (acc1,) = _sc_agg(hw1, src2, dst2, with_deg=False)
    out64 = _tc_final(hs1, acc1, deg)
    return out64[:, :C_OUT]
